# baseline (speedup 1.0000x reference)
import jax
import jax.numpy as jnp
import numpy as np
from jax.experimental import pallas as pl
from jax.experimental.pallas import tpu as pltpu

_SIZES = [256, 512, 1024, 2048]
_GRID_SIZES = [s * s for s in _SIZES]
_DIM = int(sum(_GRID_SIZES))
_BLK = 65536
_NBLK = _DIM // _BLK
_ROWS, _COLS = 512, 128
_OFFSETS = np.cumsum([0] + _GRID_SIZES)
_SEG_STARTS = (0, 1, 5, 21)
_SEG_ENDS = (0, 4, 20, 84)


def _seg_of(b):
    return ((b >= 1).astype(jnp.int32) + (b >= 5).astype(jnp.int32)
            + (b >= 21).astype(jnp.int32))


def _offset_of(seg):
    return jnp.where(seg == 0, 0,
           jnp.where(seg == 1, _OFFSETS[1],
           jnp.where(seg == 2, _OFFSETS[2], _OFFSETS[3]))).astype(jnp.int32)


def _iota2d():
    r = jax.lax.broadcasted_iota(jnp.int32, (_ROWS, _COLS), 0)
    c = jax.lax.broadcasted_iota(jnp.int32, (_ROWS, _COLS), 1)
    return r * _COLS + c


def _argmax_body(x_ref, idx_out_ref, best_val, best_idx):
    b = pl.program_id(0)
    seg = _seg_of(b)
    blk = x_ref[0]
    m = jnp.max(blk)
    loc = jnp.min(jnp.where(blk == m, _iota2d(), jnp.int32(2**30)))
    gidx = b * _BLK + loc

    is_start = (b == 0) | (b == 1) | (b == 5) | (b == 21)
    prev_val = jnp.where(is_start, jnp.float32(-jnp.inf), best_val[seg])
    prev_idx = jnp.where(is_start, jnp.int32(0), best_idx[seg])
    take = m > prev_val
    new_val = jnp.where(take, m, prev_val)
    new_idx = jnp.where(take, gidx, prev_idx)
    best_val[seg] = new_val
    best_idx[seg] = new_idx

    is_end = (b == 0) | (b == 4) | (b == 20) | (b == 84)
    @pl.when(is_end)
    def _():
        idx_out_ref[seg] = new_idx - _offset_of(seg)


def _onehot_body(idx_ref, o_ref):
    b = pl.program_id(0)
    gid = b * _BLK + _iota2d()
    hit = ((gid == idx_ref[0]) | (gid == idx_ref[1])
           | (gid == idx_ref[2]) | (gid == idx_ref[3]))
    o_ref[0] = hit.astype(jnp.float32)


def kernel(x):
    xb = x.reshape(_NBLK, _ROWS, _COLS)
    idx = pl.pallas_call(
        _argmax_body,
        grid=(_NBLK,),
        in_specs=[pl.BlockSpec((1, _ROWS, _COLS), lambda b: (b, 0, 0))],
        out_specs=pl.BlockSpec((4,), lambda b: (0,),
                               memory_space=pltpu.SMEM),
        out_shape=jax.ShapeDtypeStruct((4,), jnp.int32),
        scratch_shapes=[pltpu.SMEM((4,), jnp.float32),
                        pltpu.SMEM((4,), jnp.int32)],
    )(xb)

    onehot = pl.pallas_call(
        _onehot_body,
        grid=(_NBLK,),
        in_specs=[pl.BlockSpec(memory_space=pltpu.SMEM)],
        out_specs=pl.BlockSpec((1, _ROWS, _COLS), lambda b: (b, 0, 0)),
        out_shape=jax.ShapeDtypeStruct((_NBLK, _ROWS, _COLS), jnp.float32),
    )(idx)

    return onehot.reshape(_DIM), idx.astype(jnp.int64)


# speedup vs baseline: 1.5102x; 1.5102x over previous
import jax
import jax.numpy as jnp
import numpy as np
from jax.experimental import pallas as pl
from jax.experimental.pallas import tpu as pltpu

_SIZES = [256, 512, 1024, 2048]
_GRID_SIZES = [s * s for s in _SIZES]
_DIM = int(sum(_GRID_SIZES))
_BLK = 65536
_NBLK = _DIM // _BLK
_ROWS, _COLS = 512, 128
_OFFSETS = np.cumsum([0] + _GRID_SIZES)
_SEG_STARTS = (0, 1, 5, 21)
_SEG_ENDS = (0, 4, 20, 84)
_BIG = np.int32(2**30)


def _iota2d():
    r = jax.lax.broadcasted_iota(jnp.int32, (_ROWS, _COLS), 0)
    c = jax.lax.broadcasted_iota(jnp.int32, (_ROWS, _COLS), 1)
    return r * _COLS + c


def _main_body(x_ref, o_ref, idx_out_ref, val_accs, blk_accs):
    b = pl.program_id(0)
    o_ref[0] = jnp.zeros((_ROWS, _COLS), jnp.float32)
    blk = x_ref[0]
    for s in range(4):
        @pl.when(b == _SEG_STARTS[s])
        def _(s=s):
            val_accs[s] = blk
            blk_accs[s] = jnp.full((_ROWS, _COLS), b, jnp.int32)

        @pl.when((b > _SEG_STARTS[s]) & (b <= _SEG_ENDS[s]))
        def _(s=s):
            acc_v = val_accs[s]
            mask = blk > acc_v
            val_accs[s] = jnp.where(mask, blk, acc_v)
            blk_accs[s] = jnp.where(mask, b, blk_accs[s])

        @pl.when(b == _SEG_ENDS[s])
        def _(s=s):
            acc_v = val_accs[s]
            m = jnp.max(acc_v)
            gidx = jnp.min(jnp.where(acc_v == m,
                                     blk_accs[s] * _BLK + _iota2d(), _BIG))
            idx_out_ref[s] = gidx - np.int32(_OFFSETS[s])


def _fixup_body(idx_ref, oh_in, oh_out, row_v, sem):
    del oh_in
    for i in range(4):
        idx = idx_ref[i]
        row = idx // _COLS
        col = idx % _COLS
        cp_in = pltpu.make_async_copy(oh_out.at[pl.ds(row, 1), :], row_v, sem)
        cp_in.start()
        cp_in.wait()
        lane = jax.lax.broadcasted_iota(jnp.int32, (1, _COLS), 1)
        row_v[...] = jnp.where(lane == col, jnp.float32(1.0), row_v[...])
        cp_out = pltpu.make_async_copy(row_v, oh_out.at[pl.ds(row, 1), :], sem)
        cp_out.start()
        cp_out.wait()


def kernel(x):
    xb = x.reshape(_NBLK, _ROWS, _COLS)
    zeros2d, idx = pl.pallas_call(
        _main_body,
        grid=(_NBLK,),
        in_specs=[pl.BlockSpec((1, _ROWS, _COLS), lambda b: (b, 0, 0))],
        out_specs=[
            pl.BlockSpec((1, _ROWS, _COLS), lambda b: (b, 0, 0)),
            pl.BlockSpec((4,), lambda b: (0,), memory_space=pltpu.SMEM),
        ],
        out_shape=[
            jax.ShapeDtypeStruct((_NBLK, _ROWS, _COLS), jnp.float32),
            jax.ShapeDtypeStruct((4,), jnp.int32),
        ],
        scratch_shapes=[pltpu.VMEM((4, _ROWS, _COLS), jnp.float32),
                        pltpu.VMEM((4, _ROWS, _COLS), jnp.int32)],
    )(xb)

    onehot = pl.pallas_call(
        _fixup_body,
        in_specs=[
            pl.BlockSpec(memory_space=pltpu.SMEM),
            pl.BlockSpec(memory_space=pl.ANY),
        ],
        out_specs=pl.BlockSpec(memory_space=pl.ANY),
        out_shape=jax.ShapeDtypeStruct((_DIM // _COLS, _COLS), jnp.float32),
        scratch_shapes=[pltpu.VMEM((1, _COLS), jnp.float32),
                        pltpu.SemaphoreType.DMA],
        input_output_aliases={1: 0},
    )(idx, zeros2d.reshape(_DIM // _COLS, _COLS))

    return onehot.reshape(_DIM), idx.astype(jnp.int64)


# speedup vs baseline: 3.1035x; 2.0550x over previous
import jax
import jax.numpy as jnp
import numpy as np
from jax.experimental import pallas as pl
from jax.experimental.pallas import tpu as pltpu

_SIZES = [256, 512, 1024, 2048]
_GRID_SIZES = [s * s for s in _SIZES]
_DIM = int(sum(_GRID_SIZES))
_BLK = 65536
_NBLK = _DIM // _BLK
_ROWS, _COLS = 512, 128
_SUBS = 5
_NSTEP = _NBLK // _SUBS
_OFFSETS = np.cumsum([0] + _GRID_SIZES)
_SEG_STARTS = (0, 1, 5, 21)
_SEG_ENDS = (0, 4, 20, 84)
_BIG = np.int32(2**30)
_NCH = 8
_CROWS = _ROWS // _NCH


def _main_body(x_ref, o_ref, idx_out_ref, acc_v_ref, acc_p_ref):
    b = pl.program_id(0)
    o_ref[...] = jnp.zeros((_SUBS, _ROWS, _COLS), jnp.float32)
    for r in range(_SUBS):
        g = _SUBS * b + r
        segs = (0, 2, 3) if r == 0 else (1, 2, 3)
        for s in segs:
            @pl.when(g == _SEG_STARTS[s])
            def _(s=s):
                acc_v_ref[s] = jnp.full((_CROWS, _COLS), -jnp.inf, jnp.float32)
                acc_p_ref[s] = jnp.zeros((_CROWS, _COLS), jnp.int32)

            @pl.when((g >= _SEG_STARTS[s]) & (g <= _SEG_ENDS[s]))
            def _(r=r, s=s, g=g):
                av = acc_v_ref[s]
                ap = acc_p_ref[s]
                for k in range(_NCH):
                    chunk = x_ref[r, pl.ds(k * _CROWS, _CROWS), :]
                    mask = chunk > av
                    base = g * _BLK + k * (_CROWS * _COLS)
                    av = jnp.where(mask, chunk, av)
                    ap = jnp.where(mask, base, ap)
                acc_v_ref[s] = av
                acc_p_ref[s] = ap

            @pl.when(g == _SEG_ENDS[s])
            def _(s=s):
                av = acc_v_ref[s]
                ap = acc_p_ref[s]
                m = jnp.max(av)
                ii = jax.lax.broadcasted_iota(jnp.int32, (_CROWS, _COLS), 0)
                jj = jax.lax.broadcasted_iota(jnp.int32, (_CROWS, _COLS), 1)
                pos = jnp.min(jnp.where(av == m, ap + ii * _COLS + jj, _BIG))
                idx_out_ref[s] = pos - np.int32(_OFFSETS[s])


def _fixup_body(idx_ref, oh_in, oh_out, row_v, sem):
    del oh_in
    for i in range(4):
        idx = idx_ref[i]
        row = idx // _COLS
        col = idx % _COLS
        cp_in = pltpu.make_async_copy(oh_out.at[pl.ds(row, 1), :], row_v, sem)
        cp_in.start()
        cp_in.wait()
        lane = jax.lax.broadcasted_iota(jnp.int32, (1, _COLS), 1)
        row_v[...] = jnp.where(lane == col, jnp.float32(1.0), row_v[...])
        cp_out = pltpu.make_async_copy(row_v, oh_out.at[pl.ds(row, 1), :], sem)
        cp_out.start()
        cp_out.wait()


def kernel(x):
    xb = x.reshape(_NBLK, _ROWS, _COLS)
    zeros2d, idx = pl.pallas_call(
        _main_body,
        grid=(_NSTEP,),
        in_specs=[pl.BlockSpec((_SUBS, _ROWS, _COLS), lambda b: (b, 0, 0))],
        out_specs=[
            pl.BlockSpec((_SUBS, _ROWS, _COLS), lambda b: (b, 0, 0)),
            pl.BlockSpec((4,), lambda b: (0,), memory_space=pltpu.SMEM),
        ],
        out_shape=[
            jax.ShapeDtypeStruct((_NBLK, _ROWS, _COLS), jnp.float32),
            jax.ShapeDtypeStruct((4,), jnp.int32),
        ],
        scratch_shapes=[pltpu.VMEM((4, _CROWS, _COLS), jnp.float32),
                        pltpu.VMEM((4, _CROWS, _COLS), jnp.int32)],
    )(xb)

    onehot = pl.pallas_call(
        _fixup_body,
        in_specs=[
            pl.BlockSpec(memory_space=pltpu.SMEM),
            pl.BlockSpec(memory_space=pl.ANY),
        ],
        out_specs=pl.BlockSpec(memory_space=pl.ANY),
        out_shape=jax.ShapeDtypeStruct((_DIM // _COLS, _COLS), jnp.float32),
        scratch_shapes=[pltpu.VMEM((1, _COLS), jnp.float32),
                        pltpu.SemaphoreType.DMA],
        input_output_aliases={1: 0},
    )(idx, zeros2d.reshape(_DIM // _COLS, _COLS))

    return onehot.reshape(_DIM), idx.astype(jnp.int64)
